# baseline (speedup 1.0000x reference)
import functools

import jax
import jax.numpy as jnp
from jax import lax
from jax.experimental import pallas as pl
from jax.experimental.pallas import tpu as pltpu
from jax.experimental.pallas import tpu_sc as plsc


def _sc_gather(table, idx):
    V, D = table.shape
    B = idx.shape[0]
    info = plsc.get_sparse_core_info()
    NC, NS = info.num_cores, info.num_subcores
    NW = NC * NS
    b_per_w = B // NW
    chunk = 64
    n_chunks = b_per_w // chunk
    mesh = plsc.VectorSubcoreMesh(core_axis_name="c", subcore_axis_name="s")

    @functools.partial(
        pl.kernel,
        mesh=mesh,
        out_type=jax.ShapeDtypeStruct((B, D), jnp.float32),
        scratch_types=[
            pltpu.VMEM((chunk,), jnp.int32),
            pltpu.VMEM((chunk, D), jnp.float32),
            pltpu.SemaphoreType.DMA,
        ],
    )
    def gather_kernel(table_hbm, idx_hbm, out_hbm, idx_v, rows_v, sem):
        wid = lax.axis_index("s") * NC + lax.axis_index("c")
        for c in range(n_chunks):
            base = wid * b_per_w + c * chunk
            pltpu.sync_copy(idx_hbm.at[pl.ds(base, chunk)], idx_v)
            pltpu.async_copy(table_hbm.at[idx_v], rows_v, sem).wait()
            pltpu.sync_copy(rows_v, out_hbm.at[pl.ds(base, chunk)])

    return gather_kernel(table, idx)


def _mlp_body(h0_ref, w1_ref, b1_ref, w2_ref, b2_ref, w3_ref, b3_ref, out_ref):
    bf = jnp.bfloat16
    h = jnp.maximum(h0_ref[...], 0.0).astype(bf)
    h = jnp.dot(h, w1_ref[...], preferred_element_type=jnp.float32) + b1_ref[...]
    h = jnp.maximum(h, 0.0).astype(bf)
    h = jnp.dot(h, w2_ref[...], preferred_element_type=jnp.float32) + b2_ref[...]
    h = jnp.maximum(h, 0.0).astype(bf)
    out_ref[...] = (
        jnp.dot(h, w3_ref[...], preferred_element_type=jnp.float32) + b3_ref[...]
    )


def _mlp(h0, W1, b1, W2, b2, W3, b3):
    B, H = h0.shape
    C = W3.shape[1]
    BB = 512
    return pl.pallas_call(
        _mlp_body,
        grid=(B // BB,),
        in_specs=[
            pl.BlockSpec((BB, H), lambda i: (i, 0)),
            pl.BlockSpec((H, H), lambda i: (0, 0)),
            pl.BlockSpec((1, H), lambda i: (0, 0)),
            pl.BlockSpec((H, H), lambda i: (0, 0)),
            pl.BlockSpec((1, H), lambda i: (0, 0)),
            pl.BlockSpec((H, C), lambda i: (0, 0)),
            pl.BlockSpec((1, C), lambda i: (0, 0)),
        ],
        out_specs=pl.BlockSpec((BB, C), lambda i: (i, 0)),
        out_shape=jax.ShapeDtypeStruct((B, C), jnp.float32),
    )(
        h0,
        W1.astype(jnp.bfloat16),
        b1.reshape(1, H),
        W2.astype(jnp.bfloat16),
        b2.reshape(1, H),
        W3.astype(jnp.bfloat16),
        b3.reshape(1, C),
    )


def kernel(input, offsets, table, W1, b1, W2, b2, W3, b3):
    h0 = _sc_gather(table, input)
    return _mlp(h0, W1, b1, W2, b2, W3, b3)


# speedup vs baseline: 1.0057x; 1.0057x over previous
import functools

import jax
import jax.numpy as jnp
from jax import lax
from jax.experimental import pallas as pl
from jax.experimental.pallas import tpu as pltpu
from jax.experimental.pallas import tpu_sc as plsc


def _sc_gather(table, idx):
    V, D = table.shape
    B = idx.shape[0]
    info = plsc.get_sparse_core_info()
    NC, NS = info.num_cores, info.num_subcores
    NW = NC * NS
    b_per_w = B // NW
    chunk = 64
    n_chunks = b_per_w // chunk
    mesh = plsc.VectorSubcoreMesh(core_axis_name="c", subcore_axis_name="s")

    @functools.partial(
        pl.kernel,
        mesh=mesh,
        out_type=jax.ShapeDtypeStruct((B, D), jnp.float32),
        scratch_types=[
            pltpu.VMEM((chunk,), jnp.int32),
            pltpu.VMEM((chunk, D), jnp.float32),
            pltpu.SemaphoreType.DMA,
        ],
    )
    def gather_kernel(table_hbm, idx_hbm, out_hbm, idx_v, rows_v, sem):
        wid = lax.axis_index("s") * NC + lax.axis_index("c")
        for c in range(n_chunks):
            base = wid * b_per_w + c * chunk
            pltpu.sync_copy(idx_hbm.at[pl.ds(base, chunk)], idx_v)
            pltpu.async_copy(table_hbm.at[idx_v], rows_v, sem).wait()
            pltpu.sync_copy(rows_v, out_hbm.at[pl.ds(base, chunk)])

    return gather_kernel(table, idx)


def _mlp_body(
    h0_ref, w1_ref, b1_ref, w2_ref, b2_ref, w3_ref, b3_ref, out_hbm, acc_ref, sem
):
    i = pl.program_id(0)
    n = pl.num_programs(0)
    BB = h0_ref.shape[0]
    slot = lax.rem(i, 2)

    @pl.when(i >= 2)
    def _wait_prev():
        pltpu.make_async_copy(
            acc_ref.at[slot], out_hbm.at[pl.ds((i - 2) * BB, BB)], sem.at[slot]
        ).wait()

    bf = jnp.bfloat16
    h = jnp.maximum(h0_ref[...], 0.0).astype(bf)
    h = jnp.dot(h, w1_ref[...], preferred_element_type=jnp.float32) + b1_ref[...]
    h = jnp.maximum(h, 0.0).astype(bf)
    h = jnp.dot(h, w2_ref[...], preferred_element_type=jnp.float32) + b2_ref[...]
    h = jnp.maximum(h, 0.0).astype(bf)
    acc_ref[slot] = (
        jnp.dot(h, w3_ref[...], preferred_element_type=jnp.float32) + b3_ref[...]
    )
    pltpu.make_async_copy(
        acc_ref.at[slot], out_hbm.at[pl.ds(i * BB, BB)], sem.at[slot]
    ).start()

    @pl.when(i == n - 1)
    def _drain():
        @pl.when(n >= 2)
        def _wait_other():
            pltpu.make_async_copy(
                acc_ref.at[1 - slot],
                out_hbm.at[pl.ds((i - 1) * BB, BB)],
                sem.at[1 - slot],
            ).wait()

        pltpu.make_async_copy(
            acc_ref.at[slot], out_hbm.at[pl.ds(i * BB, BB)], sem.at[slot]
        ).wait()


def _mlp(h0, W1, b1, W2, b2, W3, b3):
    B, H = h0.shape
    C = W3.shape[1]
    BB = 512
    return pl.pallas_call(
        _mlp_body,
        grid=(B // BB,),
        in_specs=[
            pl.BlockSpec((BB, H), lambda i: (i, 0)),
            pl.BlockSpec((H, H), lambda i: (0, 0)),
            pl.BlockSpec((1, H), lambda i: (0, 0)),
            pl.BlockSpec((H, H), lambda i: (0, 0)),
            pl.BlockSpec((1, H), lambda i: (0, 0)),
            pl.BlockSpec((H, C), lambda i: (0, 0)),
            pl.BlockSpec((1, C), lambda i: (0, 0)),
        ],
        out_specs=pl.BlockSpec(memory_space=pl.ANY),
        out_shape=jax.ShapeDtypeStruct((B, C), jnp.float32),
        scratch_shapes=[
            pltpu.VMEM((2, BB, C), jnp.float32),
            pltpu.SemaphoreType.DMA((2,)),
        ],
    )(
        h0,
        W1.astype(jnp.bfloat16),
        b1.reshape(1, H),
        W2.astype(jnp.bfloat16),
        b2.reshape(1, H),
        W3.astype(jnp.bfloat16),
        b3.reshape(1, C),
    )


def kernel(input, offsets, table, W1, b1, W2, b2, W3, b3):
    h0 = _sc_gather(table, input)
    return _mlp(h0, W1, b1, W2, b2, W3, b3)


# speedup vs baseline: 1.2388x; 1.2318x over previous
import functools

import jax
import jax.numpy as jnp
from jax import lax
from jax.experimental import pallas as pl
from jax.experimental.pallas import tpu as pltpu
from jax.experimental.pallas import tpu_sc as plsc


def _sc_gather(table, idx):
    V, D = table.shape
    B = idx.shape[0]
    info = plsc.get_sparse_core_info()
    NC, NS = info.num_cores, info.num_subcores
    NW = NC * NS
    b_per_w = B // NW
    chunk = 64
    n_chunks = b_per_w // chunk
    mesh = plsc.VectorSubcoreMesh(core_axis_name="c", subcore_axis_name="s")

    @functools.partial(
        pl.kernel,
        mesh=mesh,
        out_type=jax.ShapeDtypeStruct((B, D), jnp.float32),
        scratch_types=[
            pltpu.VMEM((chunk,), jnp.int32),
            pltpu.VMEM((chunk, D), jnp.float32),
            pltpu.SemaphoreType.DMA,
        ],
    )
    def gather_kernel(table_hbm, idx_hbm, out_hbm, idx_v, rows_v, sem):
        wid = lax.axis_index("s") * NC + lax.axis_index("c")
        for c in range(n_chunks):
            base = wid * b_per_w + c * chunk
            pltpu.sync_copy(idx_hbm.at[pl.ds(base, chunk)], idx_v)
            pltpu.async_copy(table_hbm.at[idx_v], rows_v, sem).wait()
            pltpu.sync_copy(rows_v, out_hbm.at[pl.ds(base, chunk)])

    return gather_kernel(table, idx)


def _mlp_body(
    h0_ref, w1_ref, b1_ref, w2_ref, b2_ref, w3t_ref, b3_ref, out_hbm, acc_ref, sem
):
    i = pl.program_id(0)
    n = pl.num_programs(0)
    BB = h0_ref.shape[0]
    slot = lax.rem(i, 2)

    @pl.when(i >= 2)
    def _wait_prev():
        pltpu.make_async_copy(
            acc_ref.at[slot], out_hbm.at[:, pl.ds((i - 2) * BB, BB)], sem.at[slot]
        ).wait()

    bf = jnp.bfloat16
    h = jnp.maximum(h0_ref[...], 0.0).astype(bf)
    h = jnp.dot(h, w1_ref[...], preferred_element_type=jnp.float32) + b1_ref[...]
    h = jnp.maximum(h, 0.0).astype(bf)
    h = jnp.dot(h, w2_ref[...], preferred_element_type=jnp.float32) + b2_ref[...]
    h = jnp.maximum(h, 0.0).astype(bf)
    acc_ref[slot] = (
        jax.lax.dot_general(
            w3t_ref[...], h, (((1,), (1,)), ((), ())),
            preferred_element_type=jnp.float32,
        )
        + b3_ref[...]
    )
    pltpu.make_async_copy(
        acc_ref.at[slot], out_hbm.at[:, pl.ds(i * BB, BB)], sem.at[slot]
    ).start()

    @pl.when(i == n - 1)
    def _drain():
        @pl.when(n >= 2)
        def _wait_other():
            pltpu.make_async_copy(
                acc_ref.at[1 - slot],
                out_hbm.at[:, pl.ds((i - 1) * BB, BB)],
                sem.at[1 - slot],
            ).wait()

        pltpu.make_async_copy(
            acc_ref.at[slot], out_hbm.at[:, pl.ds(i * BB, BB)], sem.at[slot]
        ).wait()


def _mlp(h0, W1, b1, W2, b2, W3, b3):
    B, H = h0.shape
    C = W3.shape[1]
    BB = 512
    outT = pl.pallas_call(
        _mlp_body,
        grid=(B // BB,),
        in_specs=[
            pl.BlockSpec((BB, H), lambda i: (i, 0)),
            pl.BlockSpec((H, H), lambda i: (0, 0)),
            pl.BlockSpec((1, H), lambda i: (0, 0)),
            pl.BlockSpec((H, H), lambda i: (0, 0)),
            pl.BlockSpec((1, H), lambda i: (0, 0)),
            pl.BlockSpec((C, H), lambda i: (0, 0)),
            pl.BlockSpec((C, 1), lambda i: (0, 0)),
        ],
        out_specs=pl.BlockSpec(memory_space=pl.ANY),
        out_shape=jax.ShapeDtypeStruct((C, B), jnp.float32),
        scratch_shapes=[
            pltpu.VMEM((2, C, BB), jnp.float32),
            pltpu.SemaphoreType.DMA((2,)),
        ],
    )(
        h0,
        W1.astype(jnp.bfloat16),
        b1.reshape(1, H),
        W2.astype(jnp.bfloat16),
        b2.reshape(1, H),
        W3.T.astype(jnp.bfloat16),
        b3.reshape(C, 1),
    )
    return outT.T


def kernel(input, offsets, table, W1, b1, W2, b2, W3, b3):
    h0 = _sc_gather(table, input)
    return _mlp(h0, W1, b1, W2, b2, W3, b3)
